# baseline (speedup 1.0000x reference)
import jax
import jax.numpy as jnp
from jax import lax
from jax.experimental import pallas as pl
from jax.experimental.pallas import tpu as pltpu
from jax.experimental.pallas import tpu_sc as plsc

N = 10000
FD = 128
NC = 2
NS = 16
CHUNK = 128
KB = 8
NB0 = 7
NB1 = 13
NPAD = N + 112
DUMP = N
RPT = NPAD // NS
EPS = 1e-5


def _core_blocks(cid, sid):
    base = jnp.where(cid == 0, sid * NB0, NS * NB0 + sid * NB1)
    nb = jnp.where(cid == 0, NB0, NB1)
    return base, nb


def _deg_body(dst_hbm, out_hbm, acc, dsti, ones_v, zeros_v):
    cid = lax.axis_index("c")
    sid = lax.axis_index("s")
    wid = cid * NS + sid
    nb = dst_hbm.shape[1]

    @pl.loop(0, CHUNK)
    def _(i):
        ones_v[i] = jnp.ones((16,), jnp.float32)

    @pl.loop(0, 8)
    def _(i):
        zeros_v[i] = jnp.zeros((16,), jnp.float32)

    @pl.loop(0, RPT // 8)
    def _(i):
        pltpu.sync_copy(zeros_v, acc.at[pl.ds(sid * RPT + i * 8, 8)])

    plsc.subcore_barrier()
    pltpu.sync_copy(dst_hbm.at[wid], dsti)

    @pl.loop(0, nb)
    def _(b):
        for k in range(KB):
            pltpu.sync_copy(ones_v, acc.at[dsti.at[b, k]], add=True)

    plsc.subcore_barrier()
    pltpu.sync_copy(acc.at[pl.ds(sid * RPT, RPT)],
                    out_hbm.at[cid, pl.ds(sid * RPT, RPT)])


def _agg_body(g_hbm, src_hbm, dst_hbm, out_hbm, acc, srci, dsti, rows):
    cid = lax.axis_index("c")
    sid = lax.axis_index("s")
    base, nb = _core_blocks(cid, sid)
    last = NS - 1

    @pl.when(sid < last)
    def _():
        pltpu.sync_copy(g_hbm.at[pl.ds(sid * RPT, RPT)],
                        acc.at[pl.ds(sid * RPT, RPT)])

    @pl.when(sid == last)
    def _():
        r0 = last * RPT
        pltpu.sync_copy(g_hbm.at[pl.ds(r0, N - r0)], acc.at[pl.ds(r0, N - r0)])
        pltpu.sync_copy(g_hbm.at[pl.ds(0, NPAD - N)], acc.at[pl.ds(N, NPAD - N)])

    plsc.subcore_barrier()

    def run(nbc):
        pltpu.sync_copy(src_hbm.at[pl.ds(base, nbc)], srci.at[pl.ds(0, nbc)])
        pltpu.sync_copy(dst_hbm.at[pl.ds(base, nbc)], dsti.at[pl.ds(0, nbc)])

        @pl.loop(0, nbc)
        def _(b):
            for k in range(KB):
                pltpu.sync_copy(g_hbm.at[srci.at[b, k]], rows)
                pltpu.sync_copy(rows, acc.at[dsti.at[b, k]], add=True)

    @pl.when(cid == 0)
    def _():
        run(NB0)

    @pl.when(cid == 1)
    def _():
        run(NB1)

    plsc.subcore_barrier()
    pltpu.sync_copy(acc.at[pl.ds(sid * RPT, RPT)],
                    out_hbm.at[cid, pl.ds(sid * RPT, RPT)])


def _sc_mesh():
    return plsc.VectorSubcoreMesh(core_axis_name="c", subcore_axis_name="s",
                                  num_cores=NC, num_subcores=NS)


def _deg_call(dst_rd):
    nb = dst_rd.shape[1]
    f = pl.kernel(
        _deg_body,
        out_type=jax.ShapeDtypeStruct((NC, NPAD, 16), jnp.float32),
        mesh=_sc_mesh(),
        scratch_types=[
            pltpu.VMEM_SHARED((NPAD, 16), jnp.float32),
            pltpu.VMEM((nb, KB, CHUNK), jnp.int32),
            pltpu.VMEM((CHUNK, 16), jnp.float32),
            pltpu.VMEM((8, 16), jnp.float32),
        ],
    )
    return f(dst_rd)


def _agg_call(g, src_r, dst_r):
    f = pl.kernel(
        _agg_body,
        out_type=jax.ShapeDtypeStruct((NC, NPAD, FD), jnp.float32),
        mesh=_sc_mesh(),
        scratch_types=[
            pltpu.VMEM_SHARED((NPAD, FD), jnp.float32),
            pltpu.VMEM((max(NB0, NB1), KB, CHUNK), jnp.int32),
            pltpu.VMEM((max(NB0, NB1), KB, CHUNK), jnp.int32),
            pltpu.VMEM((CHUNK, FD), jnp.float32),
        ],
    )
    return f(g, src_r, dst_r)


def _dinv_from(degp_ref):
    deg = degp_ref[0, :N, 0:1]
    for c in range(1, NC):
        deg = deg + degp_ref[c, :N, 0:1]
    return 1.0 / jnp.sqrt(deg + 1.0)


def _acc_sum(accp_ref, g):
    s = accp_ref[0, :N, :]
    for c in range(1, NC):
        s = s + accp_ref[c, :N, :]
    if NC > 1:
        s = s - (NC - 1.0) * g
    return s


def _acc_row(accp_ref, g_ref, c):
    s = accp_ref[0, pl.ds(c, 1), :]
    for i in range(1, NC):
        s = s + accp_ref[i, pl.ds(c, 1), :]
    if NC > 1:
        s = s - (NC - 1.0) * g_ref[pl.ds(c, 1), :]
    return s


def _prescale_body(x_ref, w_ref, degp_ref, g_ref):
    h = jnp.dot(x_ref[...], w_ref[...], preferred_element_type=jnp.float32)
    g_ref[...] = h * _dinv_from(degp_ref)


def _mid_body(accp_ref, g_ref, degp_ref, b_ref, w2_ref, g2_ref):
    dinv = _dinv_from(degp_ref)
    t = dinv * _acc_sum(accp_ref, g_ref[...]) + b_ref[...]
    mean = jnp.mean(t, axis=0, keepdims=True)
    var = jnp.mean((t - mean) ** 2, axis=0, keepdims=True)
    hbn = jnp.maximum((t - mean) * lax.rsqrt(var + EPS), 0.0)
    g2_ref[...] = jnp.dot(hbn, w2_ref[...],
                          preferred_element_type=jnp.float32) * dinv


def _final_body(curr_ref, accp_ref, g_ref, degp_ref, b_ref, part_ref,
                ps_ref, h_ref):
    dinv = _dinv_from(degp_ref)
    t = dinv * _acc_sum(accp_ref, g_ref[...]) + b_ref[...]
    mean = jnp.mean(t, axis=0, keepdims=True)
    var = jnp.mean((t - mean) ** 2, axis=0, keepdims=True)
    rs = lax.rsqrt(var + EPS)
    h = jnp.maximum((t - mean) * rs, 0.0)
    h_ref[...] = h
    c = curr_ref[0]
    degc = degp_ref[0, pl.ds(c, 1), 0:1]
    for i in range(1, NC):
        degc = degc + degp_ref[i, pl.ds(c, 1), 0:1]
    dinvc = 1.0 / jnp.sqrt(degc + 1.0)
    tcr = dinvc * _acc_row(accp_ref, g_ref, c) + b_ref[...]
    xc = jnp.maximum((tcr - mean) * rs, 0.0)
    scores = jnp.sum(h * xc, axis=1, keepdims=True)
    ps_ref[...] = jnp.sum(scores * part_ref[...], axis=0, keepdims=True)


def kernel(x, edge_index, curr_node_id, partitions, node_weights,
           W1, b1, W2, b2):
    E = edge_index.shape[1]
    tb = NS * (NB0 + NB1)
    pad = tb * KB * CHUNK - E
    src = jnp.concatenate([edge_index[0], jnp.zeros((pad,), jnp.int32)])
    dst = jnp.concatenate([edge_index[1], jnp.full((pad,), DUMP, jnp.int32)])
    src_r = src.reshape(tb, KB, CHUNK)
    dst_r = dst.reshape(tb, KB, CHUNK)

    nw = NC * NS
    ewd = -(-E // (nw * KB * CHUNK)) * KB * CHUNK
    padd = nw * ewd - E
    dstd = jnp.concatenate([edge_index[1], jnp.full((padd,), DUMP, jnp.int32)])
    dst_rd = dstd.reshape(nw, ewd // (KB * CHUNK), KB, CHUNK)

    degp = _deg_call(dst_rd)

    g1 = pl.pallas_call(
        _prescale_body,
        out_shape=jax.ShapeDtypeStruct((N, FD), jnp.float32),
    )(x, W1, degp)

    acc1 = _agg_call(g1, src_r, dst_r)

    g2 = pl.pallas_call(
        _mid_body,
        out_shape=jax.ShapeDtypeStruct((N, FD), jnp.float32),
    )(acc1, g1, degp, b1.reshape(1, FD), W2)

    acc2 = _agg_call(g2, src_r, dst_r)

    curr = jnp.asarray(curr_node_id, jnp.int32).reshape(1)
    ps, h = pl.pallas_call(
        _final_body,
        in_specs=[pl.BlockSpec(memory_space=pltpu.SMEM)] + [pl.BlockSpec()] * 5,
        out_shape=[
            jax.ShapeDtypeStruct((1, partitions.shape[1]), jnp.float32),
            jax.ShapeDtypeStruct((N, FD), jnp.float32),
        ],
    )(curr, acc2, g2, degp, b2.reshape(1, FD), partitions)
    return (ps, h)


# speedup vs baseline: 1.0663x; 1.0663x over previous
import jax
import jax.numpy as jnp
from jax import lax
from jax.experimental import pallas as pl
from jax.experimental.pallas import tpu as pltpu
from jax.experimental.pallas import tpu_sc as plsc

N = 10000
FD = 128
NC = 2
NS = 16
CHUNK = 128
KB = 8
NB0 = 10
NB1 = 10
NPAD = N + 112
DUMP = N
RPT = NPAD // NS
EPS = 1e-5


def _core_blocks(cid, sid):
    base = jnp.where(cid == 0, sid * NB0, NS * NB0 + sid * NB1)
    nb = jnp.where(cid == 0, NB0, NB1)
    return base, nb


def _deg_body(dst_hbm, out_hbm, acc, dsti, ones_v, zeros_v):
    cid = lax.axis_index("c")
    sid = lax.axis_index("s")
    wid = cid * NS + sid
    nb = dst_hbm.shape[1]

    @pl.loop(0, CHUNK)
    def _(i):
        ones_v[i] = jnp.ones((16,), jnp.float32)

    @pl.loop(0, 8)
    def _(i):
        zeros_v[i] = jnp.zeros((16,), jnp.float32)

    @pl.loop(0, RPT // 8)
    def _(i):
        pltpu.sync_copy(zeros_v, acc.at[pl.ds(sid * RPT + i * 8, 8)])

    plsc.subcore_barrier()
    pltpu.sync_copy(dst_hbm.at[wid], dsti)

    @pl.loop(0, nb)
    def _(b):
        for k in range(KB):
            pltpu.sync_copy(ones_v, acc.at[dsti.at[b, k]], add=True)

    plsc.subcore_barrier()
    pltpu.sync_copy(acc.at[pl.ds(sid * RPT, RPT)],
                    out_hbm.at[cid, pl.ds(sid * RPT, RPT)])


def _agg_body(g_hbm, src_hbm, dst_hbm, out_hbm, acc, srci, dsti, rows):
    cid = lax.axis_index("c")
    sid = lax.axis_index("s")
    base, nb = _core_blocks(cid, sid)
    last = NS - 1

    @pl.when(sid < last)
    def _():
        pltpu.sync_copy(g_hbm.at[pl.ds(sid * RPT, RPT)],
                        acc.at[pl.ds(sid * RPT, RPT)])

    @pl.when(sid == last)
    def _():
        r0 = last * RPT
        pltpu.sync_copy(g_hbm.at[pl.ds(r0, N - r0)], acc.at[pl.ds(r0, N - r0)])
        pltpu.sync_copy(g_hbm.at[pl.ds(0, NPAD - N)], acc.at[pl.ds(N, NPAD - N)])

    plsc.subcore_barrier()

    def run(nbc):
        pltpu.sync_copy(src_hbm.at[pl.ds(base, nbc)], srci.at[pl.ds(0, nbc)])
        pltpu.sync_copy(dst_hbm.at[pl.ds(base, nbc)], dsti.at[pl.ds(0, nbc)])

        @pl.loop(0, nbc)
        def _(b):
            for k in range(KB):
                pltpu.sync_copy(g_hbm.at[srci.at[b, k]], rows)
                pltpu.sync_copy(rows, acc.at[dsti.at[b, k]], add=True)

    @pl.when(cid == 0)
    def _():
        run(NB0)

    @pl.when(cid == 1)
    def _():
        run(NB1)

    plsc.subcore_barrier()
    pltpu.sync_copy(acc.at[pl.ds(sid * RPT, RPT)],
                    out_hbm.at[cid, pl.ds(sid * RPT, RPT)])


def _sc_mesh():
    return plsc.VectorSubcoreMesh(core_axis_name="c", subcore_axis_name="s",
                                  num_cores=NC, num_subcores=NS)


def _deg_call(dst_rd):
    nb = dst_rd.shape[1]
    f = pl.kernel(
        _deg_body,
        out_type=jax.ShapeDtypeStruct((NC, NPAD, 16), jnp.float32),
        mesh=_sc_mesh(),
        scratch_types=[
            pltpu.VMEM_SHARED((NPAD, 16), jnp.float32),
            pltpu.VMEM((nb, KB, CHUNK), jnp.int32),
            pltpu.VMEM((CHUNK, 16), jnp.float32),
            pltpu.VMEM((8, 16), jnp.float32),
        ],
    )
    return f(dst_rd)


def _agg_call(g, src_r, dst_r):
    f = pl.kernel(
        _agg_body,
        out_type=jax.ShapeDtypeStruct((NC, NPAD, FD), jnp.float32),
        mesh=_sc_mesh(),
        scratch_types=[
            pltpu.VMEM_SHARED((NPAD, FD), jnp.float32),
            pltpu.VMEM((max(NB0, NB1), KB, CHUNK), jnp.int32),
            pltpu.VMEM((max(NB0, NB1), KB, CHUNK), jnp.int32),
            pltpu.VMEM((CHUNK, FD), jnp.float32),
        ],
    )
    return f(g, src_r, dst_r)


def _dinv_from(degp_ref):
    deg = degp_ref[0, :N, 0:1]
    for c in range(1, NC):
        deg = deg + degp_ref[c, :N, 0:1]
    return 1.0 / jnp.sqrt(deg + 1.0)


def _acc_sum(accp_ref, g):
    s = accp_ref[0, :N, :]
    for c in range(1, NC):
        s = s + accp_ref[c, :N, :]
    if NC > 1:
        s = s - (NC - 1.0) * g
    return s


def _acc_row(accp_ref, g_ref, c):
    s = accp_ref[0, pl.ds(c, 1), :]
    for i in range(1, NC):
        s = s + accp_ref[i, pl.ds(c, 1), :]
    if NC > 1:
        s = s - (NC - 1.0) * g_ref[pl.ds(c, 1), :]
    return s


def _prescale_body(x_ref, w_ref, degp_ref, g_ref):
    h = jnp.dot(x_ref[...], w_ref[...], preferred_element_type=jnp.float32)
    g_ref[...] = h * _dinv_from(degp_ref)


def _mid_body(accp_ref, g_ref, degp_ref, b_ref, w2_ref, g2_ref):
    dinv = _dinv_from(degp_ref)
    t = dinv * _acc_sum(accp_ref, g_ref[...]) + b_ref[...]
    mean = jnp.mean(t, axis=0, keepdims=True)
    var = jnp.mean((t - mean) ** 2, axis=0, keepdims=True)
    hbn = jnp.maximum((t - mean) * lax.rsqrt(var + EPS), 0.0)
    g2_ref[...] = jnp.dot(hbn, w2_ref[...],
                          preferred_element_type=jnp.float32) * dinv


def _final_body(curr_ref, accp_ref, g_ref, degp_ref, b_ref, part_ref,
                ps_ref, h_ref):
    dinv = _dinv_from(degp_ref)
    t = dinv * _acc_sum(accp_ref, g_ref[...]) + b_ref[...]
    mean = jnp.mean(t, axis=0, keepdims=True)
    var = jnp.mean((t - mean) ** 2, axis=0, keepdims=True)
    rs = lax.rsqrt(var + EPS)
    h = jnp.maximum((t - mean) * rs, 0.0)
    h_ref[...] = h
    c = curr_ref[0]
    degc = degp_ref[0, pl.ds(c, 1), 0:1]
    for i in range(1, NC):
        degc = degc + degp_ref[i, pl.ds(c, 1), 0:1]
    dinvc = 1.0 / jnp.sqrt(degc + 1.0)
    tcr = dinvc * _acc_row(accp_ref, g_ref, c) + b_ref[...]
    xc = jnp.maximum((tcr - mean) * rs, 0.0)
    scores = jnp.sum(h * xc, axis=1, keepdims=True)
    ps_ref[...] = jnp.sum(scores * part_ref[...], axis=0, keepdims=True)


def kernel(x, edge_index, curr_node_id, partitions, node_weights,
           W1, b1, W2, b2):
    E = edge_index.shape[1]
    tb = NS * (NB0 + NB1)
    pad = tb * KB * CHUNK - E
    src = jnp.concatenate([edge_index[0], jnp.zeros((pad,), jnp.int32)])
    dst = jnp.concatenate([edge_index[1], jnp.full((pad,), DUMP, jnp.int32)])
    src_r = src.reshape(tb, KB, CHUNK)
    dst_r = dst.reshape(tb, KB, CHUNK)

    nw = NC * NS
    ewd = -(-E // (nw * KB * CHUNK)) * KB * CHUNK
    padd = nw * ewd - E
    dstd = jnp.concatenate([edge_index[1], jnp.full((padd,), DUMP, jnp.int32)])
    dst_rd = dstd.reshape(nw, ewd // (KB * CHUNK), KB, CHUNK)

    degp = _deg_call(dst_rd)

    g1 = pl.pallas_call(
        _prescale_body,
        out_shape=jax.ShapeDtypeStruct((N, FD), jnp.float32),
    )(x, W1, degp)

    acc1 = _agg_call(g1, src_r, dst_r)

    g2 = pl.pallas_call(
        _mid_body,
        out_shape=jax.ShapeDtypeStruct((N, FD), jnp.float32),
    )(acc1, g1, degp, b1.reshape(1, FD), W2)

    acc2 = _agg_call(g2, src_r, dst_r)

    curr = jnp.asarray(curr_node_id, jnp.int32).reshape(1)
    ps, h = pl.pallas_call(
        _final_body,
        in_specs=[pl.BlockSpec(memory_space=pltpu.SMEM)] + [pl.BlockSpec()] * 5,
        out_shape=[
            jax.ShapeDtypeStruct((1, partitions.shape[1]), jnp.float32),
            jax.ShapeDtypeStruct((N, FD), jnp.float32),
        ],
    )(curr, acc2, g2, degp, b2.reshape(1, FD), partitions)
    return (ps, h)


# speedup vs baseline: 1.5323x; 1.4370x over previous
import jax
import jax.numpy as jnp
from jax import lax
from jax.experimental import pallas as pl
from jax.experimental.pallas import tpu as pltpu
from jax.experimental.pallas import tpu_sc as plsc

N = 10000
FD = 128
NC = 2
NS = 16
NW = NC * NS
CHUNK = 128
NPAD = N + 112
DUMP = N
RPT = NPAD // NS
EPS = 1e-5


def _deg_body(dst_hbm, out_hbm, acc, dsti, ones_v, zeros_v):
    cid = lax.axis_index("c")
    sid = lax.axis_index("s")
    wid = cid * NS + sid
    nch = dst_hbm.shape[1]

    @pl.loop(0, CHUNK)
    def _(i):
        ones_v[i] = jnp.ones((16,), jnp.float32)

    @pl.loop(0, RPT)
    def _(i):
        zeros_v[i] = jnp.zeros((16,), jnp.float32)

    pltpu.sync_copy(zeros_v, acc.at[pl.ds(sid * RPT, RPT)])
    plsc.subcore_barrier()
    pltpu.sync_copy(dst_hbm.at[wid], dsti)

    @pl.loop(0, nch)
    def _(j):
        pltpu.sync_copy(ones_v, acc.at[dsti.at[j]], add=True)

    plsc.subcore_barrier()
    pltpu.sync_copy(acc.at[pl.ds(sid * RPT, RPT)],
                    out_hbm.at[cid, pl.ds(sid * RPT, RPT)])


def _agg_body(g_hbm, src_hbm, dst_hbm, out_hbm, acc, srci, dsti, rows):
    cid = lax.axis_index("c")
    sid = lax.axis_index("s")
    wid = cid * NS + sid
    nch = src_hbm.shape[1]
    last = NS - 1

    @pl.when(sid < last)
    def _():
        pltpu.sync_copy(g_hbm.at[pl.ds(sid * RPT, RPT)],
                        acc.at[pl.ds(sid * RPT, RPT)])

    @pl.when(sid == last)
    def _():
        r0 = last * RPT
        pltpu.sync_copy(g_hbm.at[pl.ds(r0, N - r0)], acc.at[pl.ds(r0, N - r0)])
        pltpu.sync_copy(g_hbm.at[pl.ds(0, NPAD - N)], acc.at[pl.ds(N, NPAD - N)])

    plsc.subcore_barrier()
    pltpu.sync_copy(src_hbm.at[wid], srci)
    pltpu.sync_copy(dst_hbm.at[wid], dsti)

    @pl.loop(0, nch)
    def _(j):
        pltpu.sync_copy(g_hbm.at[srci.at[j]], rows)
        pltpu.sync_copy(rows, acc.at[dsti.at[j]], add=True)

    plsc.subcore_barrier()
    pltpu.sync_copy(acc.at[pl.ds(sid * RPT, RPT)],
                    out_hbm.at[cid, pl.ds(sid * RPT, RPT)])


def _sc_mesh():
    return plsc.VectorSubcoreMesh(core_axis_name="c", subcore_axis_name="s",
                                  num_cores=NC, num_subcores=NS)


def _deg_call(dst_r):
    nch = dst_r.shape[1]
    f = pl.kernel(
        _deg_body,
        out_type=jax.ShapeDtypeStruct((NC, NPAD, 16), jnp.float32),
        mesh=_sc_mesh(),
        scratch_types=[
            pltpu.VMEM_SHARED((NPAD, 16), jnp.float32),
            pltpu.VMEM((nch, CHUNK), jnp.int32),
            pltpu.VMEM((CHUNK, 16), jnp.float32),
            pltpu.VMEM((RPT, 16), jnp.float32),
        ],
    )
    return f(dst_r)


def _agg_call(g, src_r, dst_r):
    nch = src_r.shape[1]
    f = pl.kernel(
        _agg_body,
        out_type=jax.ShapeDtypeStruct((NC, NPAD, FD), jnp.float32),
        mesh=_sc_mesh(),
        scratch_types=[
            pltpu.VMEM_SHARED((NPAD, FD), jnp.float32),
            pltpu.VMEM((nch, CHUNK), jnp.int32),
            pltpu.VMEM((nch, CHUNK), jnp.int32),
            pltpu.VMEM((CHUNK, FD), jnp.float32),
        ],
    )
    return f(g, src_r, dst_r)


def _dinv_from(degp_ref):
    deg = degp_ref[0, :N, 0:1]
    for c in range(1, NC):
        deg = deg + degp_ref[c, :N, 0:1]
    return 1.0 / jnp.sqrt(deg + 1.0)


def _acc_sum(accp_ref, g):
    s = accp_ref[0, :N, :]
    for c in range(1, NC):
        s = s + accp_ref[c, :N, :]
    if NC > 1:
        s = s - (NC - 1.0) * g
    return s


def _acc_row(accp_ref, g_ref, c):
    s = accp_ref[0, pl.ds(c, 1), :]
    for i in range(1, NC):
        s = s + accp_ref[i, pl.ds(c, 1), :]
    if NC > 1:
        s = s - (NC - 1.0) * g_ref[pl.ds(c, 1), :]
    return s


def _prescale_body(x_ref, w_ref, degp_ref, g_ref):
    h = jnp.dot(x_ref[...], w_ref[...], preferred_element_type=jnp.float32)
    g_ref[...] = h * _dinv_from(degp_ref)


def _mid_body(accp_ref, g_ref, degp_ref, b_ref, w2_ref, g2_ref):
    dinv = _dinv_from(degp_ref)
    t = dinv * _acc_sum(accp_ref, g_ref[...]) + b_ref[...]
    mean = jnp.mean(t, axis=0, keepdims=True)
    var = jnp.mean((t - mean) ** 2, axis=0, keepdims=True)
    hbn = jnp.maximum((t - mean) * lax.rsqrt(var + EPS), 0.0)
    g2_ref[...] = jnp.dot(hbn, w2_ref[...],
                          preferred_element_type=jnp.float32) * dinv


def _final_body(curr_ref, accp_ref, g_ref, degp_ref, b_ref, part_ref,
                ps_ref, h_ref):
    dinv = _dinv_from(degp_ref)
    t = dinv * _acc_sum(accp_ref, g_ref[...]) + b_ref[...]
    mean = jnp.mean(t, axis=0, keepdims=True)
    var = jnp.mean((t - mean) ** 2, axis=0, keepdims=True)
    rs = lax.rsqrt(var + EPS)
    h = jnp.maximum((t - mean) * rs, 0.0)
    h_ref[...] = h
    c = curr_ref[0]
    degc = degp_ref[0, pl.ds(c, 1), 0:1]
    for i in range(1, NC):
        degc = degc + degp_ref[i, pl.ds(c, 1), 0:1]
    dinvc = 1.0 / jnp.sqrt(degc + 1.0)
    tcr = dinvc * _acc_row(accp_ref, g_ref, c) + b_ref[...]
    xc = jnp.maximum((tcr - mean) * rs, 0.0)
    scores = jnp.sum(h * xc, axis=1, keepdims=True)
    ps_ref[...] = jnp.sum(scores * part_ref[...], axis=0, keepdims=True)


def kernel(x, edge_index, curr_node_id, partitions, node_weights,
           W1, b1, W2, b2):
    E = edge_index.shape[1]
    ew = -(-E // (NW * CHUNK)) * CHUNK
    pad = NW * ew - E
    src = jnp.concatenate([edge_index[0], jnp.zeros((pad,), jnp.int32)])
    dst = jnp.concatenate([edge_index[1], jnp.full((pad,), DUMP, jnp.int32)])
    src_r = src.reshape(NW, ew // CHUNK, CHUNK)
    dst_r = dst.reshape(NW, ew // CHUNK, CHUNK)

    degp = _deg_call(dst_r)

    g1 = pl.pallas_call(
        _prescale_body,
        out_shape=jax.ShapeDtypeStruct((N, FD), jnp.float32),
    )(x, W1, degp)

    acc1 = _agg_call(g1, src_r, dst_r)

    g2 = pl.pallas_call(
        _mid_body,
        out_shape=jax.ShapeDtypeStruct((N, FD), jnp.float32),
    )(acc1, g1, degp, b1.reshape(1, FD), W2)

    acc2 = _agg_call(g2, src_r, dst_r)

    curr = jnp.asarray(curr_node_id, jnp.int32).reshape(1)
    ps, h = pl.pallas_call(
        _final_body,
        in_specs=[pl.BlockSpec(memory_space=pltpu.SMEM)] + [pl.BlockSpec()] * 5,
        out_shape=[
            jax.ShapeDtypeStruct((1, partitions.shape[1]), jnp.float32),
            jax.ShapeDtypeStruct((N, FD), jnp.float32),
        ],
    )(curr, acc2, g2, degp, b2.reshape(1, FD), partitions)
    return (ps, h)
